# baseline (speedup 1.0000x reference)
import functools

import jax
import jax.numpy as jnp
from jax import lax
from jax.experimental import pallas as pl
from jax.experimental.pallas import tpu as pltpu
from jax.experimental.pallas import tpu_sc as plsc

D_MODEL = 768
MAXLEN = 8192
SEQ = 2 * MAXLEN - 1
NW = 32
B_PER_W = 512
C = 64
NCH = B_PER_W // C
L = 16
LAST_BASE = SEQ + 1 - 8 - B_PER_W
TAIL_BASE = LAST_BASE + B_PER_W

_mesh = plsc.VectorSubcoreMesh(core_axis_name="c", subcore_axis_name="s")


@functools.partial(
    pl.kernel,
    out_type=jax.ShapeDtypeStruct((SEQ, D_MODEL), jnp.float32),
    mesh=_mesh,
    scratch_types=[
        pltpu.VMEM((B_PER_W,), jnp.int32),
        pltpu.VMEM((C, D_MODEL), jnp.float32),
        pltpu.VMEM((C, D_MODEL), jnp.float32),
        pltpu.VMEM((L,), jnp.int32),
        pltpu.VMEM((L,), jnp.int32),
        pltpu.VMEM((L, D_MODEL), jnp.float32),
        pltpu.SemaphoreType.DMA,
        pltpu.SemaphoreType.DMA,
        pltpu.SemaphoreType.DMA,
        pltpu.SemaphoreType.DMA,
        pltpu.SemaphoreType.DMA,
    ],
)
def _pe_gather(idx_hbm, table_hbm, out_hbm, idx_v, buf0, buf1,
               tail_pos, tail_idx, tail_buf,
               gsem0, gsem1, ssem0, ssem1, tsem):
    wid = lax.axis_index("s") * 2 + lax.axis_index("c")
    base = pl.multiple_of(jnp.minimum(wid * B_PER_W, LAST_BASE), 8)
    pltpu.sync_copy(idx_hbm.at[pl.ds(base, B_PER_W)], idx_v)

    def clamp_chunk(c):
        for i in range(C // L):
            p = idx_v[pl.ds(c * C + i * L, L)]
            p = jnp.minimum(jnp.maximum(p, -MAXLEN), MAXLEN - 1) + MAXLEN
            idx_v[pl.ds(c * C + i * L, L)] = p

    clamp_chunk(0)
    bufs = (buf0, buf1)
    gsems = (gsem0, gsem1)
    ssems = (ssem0, ssem1)
    gathers = [None, None]
    stores = [None, None]

    def start_gather(c):
        b = c % 2
        gathers[b] = pltpu.async_copy(
            table_hbm.at[idx_v.at[pl.ds(c * C, C)]], bufs[b], gsems[b]
        )

    start_gather(0)
    for c in range(NCH):
        b = c % 2
        nb = (c + 1) % 2
        if c + 1 < NCH:
            if stores[nb] is not None:
                stores[nb].wait()
                stores[nb] = None
            clamp_chunk(c + 1)
            start_gather(c + 1)
        gathers[b].wait()
        stores[b] = pltpu.async_copy(
            bufs[b], out_hbm.at[pl.ds(base + c * C, C)], ssems[b]
        )

    @pl.when(wid == NW - 1)
    def _tail():
        pos = jnp.minimum(lax.iota(jnp.int32, L) + TAIL_BASE, SEQ - 1)
        tail_pos[...] = pos
        pltpu.async_copy(idx_hbm.at[tail_pos], tail_idx, tsem).wait()
        p = tail_idx[...]
        tail_idx[...] = jnp.minimum(jnp.maximum(p, -MAXLEN), MAXLEN - 1) + MAXLEN
        pltpu.async_copy(table_hbm.at[tail_idx], tail_buf, tsem).wait()
        tail_pos[...] = pos
        pltpu.async_copy(tail_buf, out_hbm.at[tail_pos], tsem).wait()

    for h in stores:
        if h is not None:
            h.wait()


def kernel(pos_seq, W_k):
    return _pe_gather(pos_seq, W_k)


# speedup vs baseline: 1.0120x; 1.0120x over previous
import functools

import jax
import jax.numpy as jnp
from jax import lax
from jax.experimental import pallas as pl
from jax.experimental.pallas import tpu as pltpu
from jax.experimental.pallas import tpu_sc as plsc

D_MODEL = 768
MAXLEN = 8192
SEQ = 2 * MAXLEN - 1
NW = 32
B_PER_W = 512
C = 64
NCH = B_PER_W // C
L = 16
LAST_BASE = SEQ + 1 - 8 - B_PER_W
TAIL_BASE = LAST_BASE + B_PER_W

_mesh = plsc.VectorSubcoreMesh(core_axis_name="c", subcore_axis_name="s")


@functools.partial(
    pl.kernel,
    out_type=jax.ShapeDtypeStruct((SEQ, D_MODEL), jnp.float32),
    mesh=_mesh,
    scratch_types=[
        pltpu.VMEM((B_PER_W,), jnp.int32),
        pltpu.VMEM((C, D_MODEL), jnp.float32),
        pltpu.VMEM((C, D_MODEL), jnp.float32),
        pltpu.VMEM((L,), jnp.int32),
        pltpu.VMEM((L,), jnp.int32),
        pltpu.VMEM((L, D_MODEL), jnp.float32),
        pltpu.SemaphoreType.DMA,
        pltpu.SemaphoreType.DMA,
        pltpu.SemaphoreType.DMA,
        pltpu.SemaphoreType.DMA,
        pltpu.SemaphoreType.DMA,
    ],
)
def _pe_gather(idx_hbm, table_hbm, out_hbm, idx_v, buf0, buf1,
               tail_pos, tail_idx, tail_buf,
               gsem0, gsem1, ssem0, ssem1, tsem):
    wid = lax.axis_index("s") * 2 + lax.axis_index("c")
    base = pl.multiple_of(jnp.minimum(wid * B_PER_W, LAST_BASE), 8)
    pltpu.sync_copy(idx_hbm.at[pl.ds(base, B_PER_W)], idx_v)

    def clamp_chunk(c):
        for i in range(C // L):
            p = idx_v[pl.ds(c * C + i * L, L)]
            p = jnp.minimum(jnp.maximum(p, -MAXLEN), MAXLEN - 1) + MAXLEN
            idx_v[pl.ds(c * C + i * L, L)] = p

    clamp_chunk(0)
    bufs = (buf0, buf1)
    gsems = (gsem0, gsem1)
    ssems = (ssem0, ssem1)
    gathers = [None, None]
    stores = [None, None]

    def start_gather(c):
        b = c % 2
        gathers[b] = pltpu.async_copy(
            table_hbm.at[idx_v.at[pl.ds(c * C, C)]], bufs[b], gsems[b]
        )

    is_last = wid == NW - 1
    tail_chain = [None]

    @pl.when(is_last)
    def _tail_stage0():
        tail_pos[...] = jnp.minimum(lax.iota(jnp.int32, L) + TAIL_BASE, SEQ - 1)
        tail_chain[0] = pltpu.async_copy(idx_hbm.at[tail_pos], tail_idx, tsem)

    def tail_stage(c):
        if c == 2:
            @pl.when(is_last)
            def _clamp_and_gather():
                tail_chain[0].wait()
                p = tail_idx[...]
                tail_idx[...] = (
                    jnp.minimum(jnp.maximum(p, -MAXLEN), MAXLEN - 1) + MAXLEN
                )
                tail_chain[0] = pltpu.async_copy(
                    table_hbm.at[tail_idx], tail_buf, tsem
                )
        elif c == 5:
            @pl.when(is_last)
            def _scatter():
                tail_chain[0].wait()
                tail_chain[0] = pltpu.async_copy(
                    tail_buf, out_hbm.at[tail_pos], tsem
                )

    start_gather(0)
    for c in range(NCH):
        b = c % 2
        nb = (c + 1) % 2
        if c + 1 < NCH:
            if stores[nb] is not None:
                stores[nb].wait()
                stores[nb] = None
            clamp_chunk(c + 1)
            start_gather(c + 1)
        gathers[b].wait()
        stores[b] = pltpu.async_copy(
            bufs[b], out_hbm.at[pl.ds(base + c * C, C)], ssems[b]
        )
        tail_stage(c)

    for h in stores:
        if h is not None:
            h.wait()

    @pl.when(is_last)
    def _tail_drain():
        tail_chain[0].wait()


def kernel(pos_seq, W_k):
    return _pe_gather(pos_seq, W_k)


# speedup vs baseline: 1.0144x; 1.0024x over previous
import functools

import jax
import jax.numpy as jnp
from jax import lax
from jax.experimental import pallas as pl
from jax.experimental.pallas import tpu as pltpu
from jax.experimental.pallas import tpu_sc as plsc

D_MODEL = 768
MAXLEN = 8192
SEQ = 2 * MAXLEN - 1
NW = 32
B_PER_W = 512
C = 32
NCH = B_PER_W // C
L = 16
LAST_BASE = SEQ + 1 - 8 - B_PER_W
TAIL_BASE = LAST_BASE + B_PER_W

_mesh = plsc.VectorSubcoreMesh(core_axis_name="c", subcore_axis_name="s")


@functools.partial(
    pl.kernel,
    out_type=jax.ShapeDtypeStruct((SEQ, D_MODEL), jnp.float32),
    mesh=_mesh,
    scratch_types=[
        pltpu.VMEM((B_PER_W,), jnp.int32),
        pltpu.VMEM((C, D_MODEL), jnp.float32),
        pltpu.VMEM((C, D_MODEL), jnp.float32),
        pltpu.VMEM((C, D_MODEL), jnp.float32),
        pltpu.VMEM((C, D_MODEL), jnp.float32),
        pltpu.VMEM((L,), jnp.int32),
        pltpu.VMEM((L,), jnp.int32),
        pltpu.VMEM((L, D_MODEL), jnp.float32),
        pltpu.SemaphoreType.DMA,
        pltpu.SemaphoreType.DMA,
        pltpu.SemaphoreType.DMA,
        pltpu.SemaphoreType.DMA,
        pltpu.SemaphoreType.DMA,
        pltpu.SemaphoreType.DMA,
        pltpu.SemaphoreType.DMA,
        pltpu.SemaphoreType.DMA,
        pltpu.SemaphoreType.DMA,
    ],
)
def _pe_gather(idx_hbm, table_hbm, out_hbm, idx_v, buf0, buf1, buf2, buf3,
               tail_pos, tail_idx, tail_buf,
               gsem0, gsem1, gsem2, gsem3, ssem0, ssem1, ssem2, ssem3, tsem):
    wid = lax.axis_index("s") * 2 + lax.axis_index("c")
    base = pl.multiple_of(jnp.minimum(wid * B_PER_W, LAST_BASE), 8)
    pltpu.sync_copy(idx_hbm.at[pl.ds(base, B_PER_W)], idx_v)

    def clamp_chunk(c):
        for i in range(C // L):
            p = idx_v[pl.ds(c * C + i * L, L)]
            p = jnp.minimum(jnp.maximum(p, -MAXLEN), MAXLEN - 1) + MAXLEN
            idx_v[pl.ds(c * C + i * L, L)] = p

    bufs = (buf0, buf1, buf2, buf3)
    gsems = (gsem0, gsem1, gsem2, gsem3)
    ssems = (ssem0, ssem1, ssem2, ssem3)
    NBUF = 4
    AHEAD = 3
    gathers = [None] * NBUF
    stores = [None] * NBUF

    def start_gather(c):
        b = c % NBUF
        gathers[b] = pltpu.async_copy(
            table_hbm.at[idx_v.at[pl.ds(c * C, C)]], bufs[b], gsems[b]
        )

    is_last = wid == NW - 1
    tail_chain = [None]

    @pl.when(is_last)
    def _tail_stage0():
        tail_pos[...] = jnp.minimum(lax.iota(jnp.int32, L) + TAIL_BASE, SEQ - 1)
        tail_chain[0] = pltpu.async_copy(idx_hbm.at[tail_pos], tail_idx, tsem)

    def tail_stage(c):
        if c == 4:
            @pl.when(is_last)
            def _clamp_and_gather():
                tail_chain[0].wait()
                p = tail_idx[...]
                tail_idx[...] = (
                    jnp.minimum(jnp.maximum(p, -MAXLEN), MAXLEN - 1) + MAXLEN
                )
                tail_chain[0] = pltpu.async_copy(
                    table_hbm.at[tail_idx], tail_buf, tsem
                )
        elif c == 10:
            @pl.when(is_last)
            def _scatter():
                tail_chain[0].wait()
                tail_chain[0] = pltpu.async_copy(
                    tail_buf, out_hbm.at[tail_pos], tsem
                )

    for k in range(AHEAD):
        clamp_chunk(k)
        start_gather(k)
    for c in range(NCH):
        b = c % NBUF
        gathers[b].wait()
        stores[b] = pltpu.async_copy(
            bufs[b], out_hbm.at[pl.ds(base + c * C, C)], ssems[b]
        )
        n = c + AHEAD
        if n < NCH:
            nb = n % NBUF
            if stores[nb] is not None:
                stores[nb].wait()
                stores[nb] = None
            clamp_chunk(n)
            start_gather(n)
        tail_stage(c)

    for h in stores:
        if h is not None:
            h.wait()

    @pl.when(is_last)
    def _tail_drain():
        tail_chain[0].wait()


def kernel(pos_seq, W_k):
    return _pe_gather(pos_seq, W_k)
